# Initial kernel scaffold; baseline (speedup 1.0000x reference)
import jax
import jax.numpy as jnp
from jax.experimental import pallas as pl


def kernel(starts, paths, ends, values_table, paths_table, W, a, W_out):
    raise NotImplementedError("write your pallas kernel here")


# speedup vs baseline: 3.7387x; 3.7387x over previous
import functools

import jax
import jax.numpy as jnp
from jax import lax
from jax.experimental import pallas as pl
from jax.experimental.pallas import tpu as pltpu
from jax.experimental.pallas import tpu_sc as plsc

B = 1024
NPATHS = 200
D = 128
LABELS = 1000
NEG_INF = -2.0 * 10**10

TOT = B * NPATHS
NC = 2
NS = 16
NW = NC * NS
PER_W = TOT // NW
CHUNK = 128
N_CHUNKS = PER_W // CHUNK


def _sc_gather_body(starts_h, paths_h, ends_h, vt_h, pt_h,
                    os_h, op_h, oe_h, idx_v, rows_v, sem):
    wid = lax.axis_index("s") * NC + lax.axis_index("c")
    base = wid * PER_W
    for idx_h, tab_h, out_h in ((starts_h, vt_h, os_h),
                                (paths_h, pt_h, op_h),
                                (ends_h, vt_h, oe_h)):
        def body(i, _):
            off = base + i * CHUNK
            pltpu.sync_copy(idx_h.at[pl.ds(off, CHUNK)], idx_v)
            pltpu.async_copy(tab_h.at[idx_v], rows_v, sem).wait()
            pltpu.sync_copy(rows_v, out_h.at[pl.ds(off, CHUNK), :])
            return 0
        lax.fori_loop(0, N_CHUNKS, body, 0)


@functools.cache
def _sc_gather():
    return functools.partial(
        pl.kernel,
        mesh=plsc.VectorSubcoreMesh(core_axis_name="c", subcore_axis_name="s"),
        out_type=(
            jax.ShapeDtypeStruct((TOT, D), jnp.float32),
            jax.ShapeDtypeStruct((TOT, D), jnp.float32),
            jax.ShapeDtypeStruct((TOT, D), jnp.float32),
        ),
        scratch_types=[
            pltpu.VMEM((CHUNK,), jnp.int32),
            pltpu.VMEM((CHUNK, D), jnp.float32),
            pltpu.SemaphoreType.DMA,
        ],
    )(_sc_gather_body)


BB = 16


def _tc_body(s_ref, p_ref, e_ref, st_ref, ws_ref, wp_ref, we_ref,
             a_ref, wo_ref, cv_ref, out_ref):
    s = s_ref[...].reshape(BB * NPATHS, D)
    p = p_ref[...].reshape(BB * NPATHS, D)
    e = e_ref[...].reshape(BB * NPATHS, D)
    acc = jnp.dot(s, ws_ref[...], preferred_element_type=jnp.float32)
    acc = acc + jnp.dot(p, wp_ref[...], preferred_element_type=jnp.float32)
    acc = acc + jnp.dot(e, we_ref[...], preferred_element_type=jnp.float32)
    comb = jnp.tanh(acc)
    a_row = a_ref[...].reshape(1, D)
    logits = jnp.sum(comb * a_row, axis=1).reshape(BB, NPATHS)
    m = (st_ref[...] > 1).astype(jnp.float32)
    z = logits * m + (1.0 - m) * NEG_INF
    zmax = jnp.max(z, axis=1, keepdims=True)
    ez = jnp.exp(z - zmax)
    w = ez / jnp.sum(ez, axis=1, keepdims=True)
    comb3 = comb.reshape(BB, NPATHS, D)
    cv = jnp.sum(comb3 * w[:, :, None], axis=1)
    cv_ref[...] = cv
    out_ref[...] = jnp.dot(cv, wo_ref[...], preferred_element_type=jnp.float32)


def _tc_dense(s_g, p_g, e_g, starts, Ws, Wp, We, a, W_out):
    grid = (B // BB,)
    return pl.pallas_call(
        _tc_body,
        grid=grid,
        in_specs=[
            pl.BlockSpec((BB, NPATHS, D), lambda i: (i, 0, 0)),
            pl.BlockSpec((BB, NPATHS, D), lambda i: (i, 0, 0)),
            pl.BlockSpec((BB, NPATHS, D), lambda i: (i, 0, 0)),
            pl.BlockSpec((BB, NPATHS), lambda i: (i, 0)),
            pl.BlockSpec((D, D), lambda i: (0, 0)),
            pl.BlockSpec((D, D), lambda i: (0, 0)),
            pl.BlockSpec((D, D), lambda i: (0, 0)),
            pl.BlockSpec((1, D), lambda i: (0, 0)),
            pl.BlockSpec((D, LABELS), lambda i: (0, 0)),
        ],
        out_specs=[
            pl.BlockSpec((BB, D), lambda i: (i, 0)),
            pl.BlockSpec((BB, LABELS), lambda i: (i, 0)),
        ],
        out_shape=[
            jax.ShapeDtypeStruct((B, D), jnp.float32),
            jax.ShapeDtypeStruct((B, LABELS), jnp.float32),
        ],
    )(s_g, p_g, e_g, starts, Ws, Wp, We, a, W_out)


def kernel(starts, paths, ends, values_table, paths_table, W, a, W_out):
    s_g, p_g, e_g = _sc_gather()(
        starts.reshape(TOT), paths.reshape(TOT), ends.reshape(TOT),
        values_table, paths_table)
    s_g = s_g.reshape(B, NPATHS, D)
    p_g = p_g.reshape(B, NPATHS, D)
    e_g = e_g.reshape(B, NPATHS, D)
    Ws, Wp, We = W[:D], W[D:2 * D], W[2 * D:]
    cv, out = _tc_dense(s_g, p_g, e_g, starts, Ws, Wp, We, a, W_out)
    return (cv, out)


# speedup vs baseline: 5.1028x; 1.3649x over previous
import functools

import jax
import jax.numpy as jnp
from jax import lax
from jax.experimental import pallas as pl
from jax.experimental.pallas import tpu as pltpu
from jax.experimental.pallas import tpu_sc as plsc

B = 1024
NPATHS = 200
D = 128
LABELS = 1000
NEG_INF = -2.0 * 10**10

TOT = B * NPATHS
NC = 2
NS = 16
NW = NC * NS
PER_W = TOT // NW
CHUNK = 128
N_CHUNKS = PER_W // CHUNK


def _sc_gather_body(starts_h, paths_h, ends_h, vt_h, pt_h,
                    os_h, op_h, oe_h,
                    idx_s, idx_p, idx_e,
                    r00, r01, r02, r10, r11, r12,
                    gsem0, gsem1, ssem0, ssem1):
    wid = lax.axis_index("s") * NC + lax.axis_index("c")
    base = wid * PER_W
    gsem = (gsem0, gsem1)
    ssem = (ssem0, ssem1)
    tabs = (vt_h, pt_h, vt_h)
    outs = (os_h, op_h, oe_h)
    idxs = (idx_s, idx_p, idx_e)
    rows = ((r00, r01, r02), (r10, r11, r12))

    for t, idx_h in enumerate((starts_h, paths_h, ends_h)):
        pltpu.sync_copy(idx_h.at[pl.ds(base, PER_W)], idxs[t])

    def issue_gathers(slot, g):
        for t in range(3):
            pltpu.async_copy(
                tabs[t].at[idxs[t].at[pl.ds(g * CHUNK, CHUNK)]],
                rows[slot][t], gsem[slot])

    def wait_gathers(slot):
        for t in range(3):
            pltpu.make_async_copy(
                tabs[t].at[pl.ds(0, CHUNK), :],
                rows[slot][t], gsem[slot]).wait()

    def issue_stores(slot, g):
        for t in range(3):
            pltpu.async_copy(
                rows[slot][t],
                outs[t].at[pl.ds(base + g * CHUNK, CHUNK), :], ssem[slot])

    def wait_stores(slot):
        for t in range(3):
            pltpu.make_async_copy(
                rows[slot][t],
                outs[t].at[pl.ds(0, CHUNK), :], ssem[slot]).wait()

    issue_gathers(0, 0)
    issue_gathers(1, 1)

    def body(j, _):
        g0 = 2 * j
        g1 = g0 + 1
        wait_gathers(0)
        issue_stores(0, g0)
        wait_gathers(1)
        issue_stores(1, g1)
        wait_stores(0)

        @pl.when(g0 + 2 < N_CHUNKS)
        def _():
            issue_gathers(0, g0 + 2)
        wait_stores(1)

        @pl.when(g1 + 2 < N_CHUNKS)
        def _():
            issue_gathers(1, g1 + 2)
        return 0

    lax.fori_loop(0, N_CHUNKS // 2, body, 0)


@functools.cache
def _sc_gather():
    return functools.partial(
        pl.kernel,
        mesh=plsc.VectorSubcoreMesh(core_axis_name="c", subcore_axis_name="s"),
        out_type=(
            jax.ShapeDtypeStruct((TOT, D), jnp.float32),
            jax.ShapeDtypeStruct((TOT, D), jnp.float32),
            jax.ShapeDtypeStruct((TOT, D), jnp.float32),
        ),
        scratch_types=(
            [pltpu.VMEM((PER_W,), jnp.int32)] * 3
            + [pltpu.VMEM((CHUNK, D), jnp.float32)] * 6
            + [pltpu.SemaphoreType.DMA] * 4
        ),
    )(_sc_gather_body)


BB = 16


def _tc_body(s_ref, p_ref, e_ref, st_ref, ws_ref, wp_ref, we_ref,
             a_ref, wo_ref, cv_ref, out_ref):
    s = s_ref[...].reshape(BB * NPATHS, D)
    p = p_ref[...].reshape(BB * NPATHS, D)
    e = e_ref[...].reshape(BB * NPATHS, D)
    acc = jnp.dot(s, ws_ref[...], preferred_element_type=jnp.float32)
    acc = acc + jnp.dot(p, wp_ref[...], preferred_element_type=jnp.float32)
    acc = acc + jnp.dot(e, we_ref[...], preferred_element_type=jnp.float32)
    comb = jnp.tanh(acc)
    a_row = a_ref[...].reshape(1, D)
    logits = jnp.sum(comb * a_row, axis=1).reshape(BB, NPATHS)
    m = (st_ref[...] > 1).astype(jnp.float32)
    z = logits * m + (1.0 - m) * NEG_INF
    zmax = jnp.max(z, axis=1, keepdims=True)
    ez = jnp.exp(z - zmax)
    w = ez / jnp.sum(ez, axis=1, keepdims=True)
    comb3 = comb.reshape(BB, NPATHS, D)
    cv = jnp.sum(comb3 * w[:, :, None], axis=1)
    cv_ref[...] = cv
    out_ref[...] = jnp.dot(cv, wo_ref[...], preferred_element_type=jnp.float32)


def _tc_dense(s_g, p_g, e_g, starts, Ws, Wp, We, a, W_out):
    grid = (B // BB,)
    return pl.pallas_call(
        _tc_body,
        grid=grid,
        in_specs=[
            pl.BlockSpec((BB, NPATHS, D), lambda i: (i, 0, 0)),
            pl.BlockSpec((BB, NPATHS, D), lambda i: (i, 0, 0)),
            pl.BlockSpec((BB, NPATHS, D), lambda i: (i, 0, 0)),
            pl.BlockSpec((BB, NPATHS), lambda i: (i, 0)),
            pl.BlockSpec((D, D), lambda i: (0, 0)),
            pl.BlockSpec((D, D), lambda i: (0, 0)),
            pl.BlockSpec((D, D), lambda i: (0, 0)),
            pl.BlockSpec((1, D), lambda i: (0, 0)),
            pl.BlockSpec((D, LABELS), lambda i: (0, 0)),
        ],
        out_specs=[
            pl.BlockSpec((BB, D), lambda i: (i, 0)),
            pl.BlockSpec((BB, LABELS), lambda i: (i, 0)),
        ],
        out_shape=[
            jax.ShapeDtypeStruct((B, D), jnp.float32),
            jax.ShapeDtypeStruct((B, LABELS), jnp.float32),
        ],
    )(s_g, p_g, e_g, starts, Ws, Wp, We, a, W_out)


def kernel(starts, paths, ends, values_table, paths_table, W, a, W_out):
    s_g, p_g, e_g = _sc_gather()(
        starts.reshape(TOT), paths.reshape(TOT), ends.reshape(TOT),
        values_table, paths_table)
    s_g = s_g.reshape(B, NPATHS, D)
    p_g = p_g.reshape(B, NPATHS, D)
    e_g = e_g.reshape(B, NPATHS, D)
    Ws, Wp, We = W[:D], W[D:2 * D], W[2 * D:]
    cv, out = _tc_dense(s_g, p_g, e_g, starts, Ws, Wp, We, a, W_out)
    return (cv, out)


# speedup vs baseline: 5.9510x; 1.1662x over previous
import functools

import jax
import jax.numpy as jnp
from jax import lax
from jax.experimental import pallas as pl
from jax.experimental.pallas import tpu as pltpu
from jax.experimental.pallas import tpu_sc as plsc

B = 1024
NPATHS = 200
D = 128
LABELS = 1000
NEG_INF = -2.0 * 10**10

NC = 2
NS = 16
NW = NC * NS

NSPLIT = 4
BC = B // NSPLIT
TOTC = BC * NPATHS
PER_W = TOTC // NW
CHUNK = 80
N_CHUNKS = PER_W // CHUNK


def _sc_gather_body(starts_h, paths_h, ends_h, vt_h, pt_h,
                    os_h, op_h, oe_h,
                    idx_s, idx_p, idx_e,
                    r00, r01, r02, r10, r11, r12,
                    gsem0, gsem1, ssem0, ssem1):
    wid = lax.axis_index("s") * NC + lax.axis_index("c")
    base = wid * PER_W
    gsem = (gsem0, gsem1)
    ssem = (ssem0, ssem1)
    tabs = (vt_h, pt_h, vt_h)
    outs = (os_h, op_h, oe_h)
    idxs = (idx_s, idx_p, idx_e)
    rows = ((r00, r01, r02), (r10, r11, r12))

    for t, idx_h in enumerate((starts_h, paths_h, ends_h)):
        pltpu.sync_copy(idx_h.at[pl.ds(base, PER_W)], idxs[t])

    def issue_gathers(slot, g):
        for t in range(3):
            pltpu.async_copy(
                tabs[t].at[idxs[t].at[pl.ds(g * CHUNK, CHUNK)]],
                rows[slot][t], gsem[slot])

    def wait_gathers(slot):
        for t in range(3):
            pltpu.make_async_copy(
                tabs[t].at[pl.ds(0, CHUNK), :],
                rows[slot][t], gsem[slot]).wait()

    def issue_stores(slot, g):
        for t in range(3):
            pltpu.async_copy(
                rows[slot][t],
                outs[t].at[pl.ds(base + g * CHUNK, CHUNK), :], ssem[slot])

    def wait_stores(slot):
        for t in range(3):
            pltpu.make_async_copy(
                rows[slot][t],
                outs[t].at[pl.ds(0, CHUNK), :], ssem[slot]).wait()

    issue_gathers(0, 0)
    issue_gathers(1, 1)

    def body(j, _):
        g0 = 2 * j
        g1 = g0 + 1
        wait_gathers(0)
        issue_stores(0, g0)
        wait_gathers(1)
        issue_stores(1, g1)
        wait_stores(0)

        @pl.when(g0 + 2 < N_CHUNKS)
        def _():
            issue_gathers(0, g0 + 2)
        wait_stores(1)

        @pl.when(g1 + 2 < N_CHUNKS)
        def _():
            issue_gathers(1, g1 + 2)
        return 0

    lax.fori_loop(0, N_CHUNKS // 2, body, 0)


@functools.cache
def _sc_gather():
    return functools.partial(
        pl.kernel,
        mesh=plsc.VectorSubcoreMesh(core_axis_name="c", subcore_axis_name="s"),
        out_type=(
            jax.ShapeDtypeStruct((TOTC, D), jnp.float32),
            jax.ShapeDtypeStruct((TOTC, D), jnp.float32),
            jax.ShapeDtypeStruct((TOTC, D), jnp.float32),
        ),
        scratch_types=(
            [pltpu.VMEM((PER_W,), jnp.int32)] * 3
            + [pltpu.VMEM((CHUNK, D), jnp.float32)] * 6
            + [pltpu.SemaphoreType.DMA] * 4
        ),
    )(_sc_gather_body)


BB = 16


def _tc_body(s_ref, p_ref, e_ref, st_ref, ws_ref, wp_ref, we_ref,
             a_ref, wo_ref, cv_ref, out_ref):
    s = s_ref[...].reshape(BB * NPATHS, D)
    p = p_ref[...].reshape(BB * NPATHS, D)
    e = e_ref[...].reshape(BB * NPATHS, D)
    acc = jnp.dot(s, ws_ref[...], preferred_element_type=jnp.float32)
    acc = acc + jnp.dot(p, wp_ref[...], preferred_element_type=jnp.float32)
    acc = acc + jnp.dot(e, we_ref[...], preferred_element_type=jnp.float32)
    comb = jnp.tanh(acc)
    a_row = a_ref[...].reshape(1, D)
    logits = jnp.sum(comb * a_row, axis=1).reshape(BB, NPATHS)
    m = (st_ref[...] > 1).astype(jnp.float32)
    z = logits * m + (1.0 - m) * NEG_INF
    zmax = jnp.max(z, axis=1, keepdims=True)
    ez = jnp.exp(z - zmax)
    w = ez / jnp.sum(ez, axis=1, keepdims=True)
    comb3 = comb.reshape(BB, NPATHS, D)
    cv = jnp.sum(comb3 * w[:, :, None], axis=1)
    cv_ref[...] = cv
    out_ref[...] = jnp.dot(cv, wo_ref[...], preferred_element_type=jnp.float32)


def _tc_dense(s_g, p_g, e_g, starts_c, Ws, Wp, We, a, W_out):
    grid = (BC // BB,)
    return pl.pallas_call(
        _tc_body,
        grid=grid,
        in_specs=[
            pl.BlockSpec((BB, NPATHS, D), lambda i: (i, 0, 0)),
            pl.BlockSpec((BB, NPATHS, D), lambda i: (i, 0, 0)),
            pl.BlockSpec((BB, NPATHS, D), lambda i: (i, 0, 0)),
            pl.BlockSpec((BB, NPATHS), lambda i: (i, 0)),
            pl.BlockSpec((D, D), lambda i: (0, 0)),
            pl.BlockSpec((D, D), lambda i: (0, 0)),
            pl.BlockSpec((D, D), lambda i: (0, 0)),
            pl.BlockSpec((1, D), lambda i: (0, 0)),
            pl.BlockSpec((D, LABELS), lambda i: (0, 0)),
        ],
        out_specs=[
            pl.BlockSpec((BB, D), lambda i: (i, 0)),
            pl.BlockSpec((BB, LABELS), lambda i: (i, 0)),
        ],
        out_shape=[
            jax.ShapeDtypeStruct((BC, D), jnp.float32),
            jax.ShapeDtypeStruct((BC, LABELS), jnp.float32),
        ],
    )(s_g, p_g, e_g, starts_c, Ws, Wp, We, a, W_out)


def kernel(starts, paths, ends, values_table, paths_table, W, a, W_out):
    starts_f = starts.reshape(B * NPATHS)
    paths_f = paths.reshape(B * NPATHS)
    ends_f = ends.reshape(B * NPATHS)
    Ws, Wp, We = W[:D], W[D:2 * D], W[2 * D:]
    sc = _sc_gather()
    cvs, outs = [], []
    for c in range(NSPLIT):
        lo = c * TOTC
        s_g, p_g, e_g = sc(
            lax.dynamic_slice_in_dim(starts_f, lo, TOTC),
            lax.dynamic_slice_in_dim(paths_f, lo, TOTC),
            lax.dynamic_slice_in_dim(ends_f, lo, TOTC),
            values_table, paths_table)
        s_g = s_g.reshape(BC, NPATHS, D)
        p_g = p_g.reshape(BC, NPATHS, D)
        e_g = e_g.reshape(BC, NPATHS, D)
        starts_c = lax.dynamic_slice_in_dim(starts, c * BC, BC)
        cv, out = _tc_dense(s_g, p_g, e_g, starts_c, Ws, Wp, We, a, W_out)
        cvs.append(cv)
        outs.append(out)
    return (jnp.concatenate(cvs, axis=0), jnp.concatenate(outs, axis=0))
